# baseline (speedup 1.0000x reference)
import functools

import jax
import jax.numpy as jnp
from jax import lax
from jax.experimental import pallas as pl
from jax.experimental.pallas import tpu as pltpu
from jax.experimental.pallas import tpu_sc as plsc

D_MODEL = 64
SCALE = 8.0


@functools.lru_cache(maxsize=None)
def _make_lookup(B, V, D):
    info = plsc.get_sparse_core_info()
    NC, NS, L = info.num_cores, info.num_subcores, info.num_lanes
    NW = NC * NS
    assert B % NW == 0 and D % L == 0
    b_per_w = B // NW
    C = 512
    NBUF = 2
    assert b_per_w % C == 0
    n_chunks = b_per_w // C
    assert n_chunks % NBUF == 0
    n_outer = n_chunks // NBUF
    n_vregs = D // L

    mesh = plsc.VectorSubcoreMesh(core_axis_name="c", subcore_axis_name="s")

    @functools.partial(
        pl.kernel,
        mesh=mesh,
        compiler_params=pltpu.CompilerParams(use_tc_tiling_on_sc=False),
        out_type=jax.ShapeDtypeStruct((B, D), jnp.float32),
        scratch_types=[
            tuple(pltpu.VMEM((C,), jnp.int32) for _ in range(NBUF)),
            tuple(pltpu.VMEM((C, D), jnp.float32) for _ in range(NBUF)),
            tuple(pltpu.SemaphoreType.DMA for _ in range(NBUF)),
            tuple(pltpu.SemaphoreType.DMA for _ in range(NBUF)),
        ],
    )
    def lookup(x_hbm, table_hbm, out_hbm, idx_bufs, row_bufs, gsems, ssems):
        wid = lax.axis_index("s") * NC + lax.axis_index("c")
        base = wid * b_per_w

        def issue_gather(c, b):
            pltpu.sync_copy(x_hbm.at[pl.ds(base + c * C, C)], idx_bufs[b])
            pltpu.make_async_copy(
                table_hbm.at[idx_bufs[b]], row_bufs[b], gsems[b]
            ).start()

        for b in range(NBUF):
            issue_gather(b, b)

        def outer(i, carry):
            for b in range(NBUF):
                c = i * NBUF + b
                pltpu.make_async_copy(
                    table_hbm.at[idx_bufs[b]], row_bufs[b], gsems[b]
                ).wait()

                def scale_rows(r, carry2):
                    for j in range(n_vregs):
                        row_bufs[b][r, pl.ds(j * L, L)] = (
                            row_bufs[b][r, pl.ds(j * L, L)] * SCALE
                        )
                    return carry2

                lax.fori_loop(0, C, scale_rows, 0, unroll=4)

                store = pltpu.make_async_copy(
                    row_bufs[b],
                    out_hbm.at[pl.ds(base + c * C, C)],
                    ssems[b],
                )
                store.start()

                @pl.when(c + NBUF < n_chunks)
                def _():
                    store.wait()
                    issue_gather(c + NBUF, b)

            return carry

        lax.fori_loop(0, n_outer, outer, 0)

        for b in range(NBUF):
            pltpu.make_async_copy(
                row_bufs[b], out_hbm.at[pl.ds(base, C)], ssems[b]
            ).wait()

    return lookup


def kernel(x, table):
    B = x.shape[0] * x.shape[1]
    V, D = table.shape
    out = _make_lookup(B, V, D)(x.reshape(B), table)
    return out.reshape(x.shape[0], x.shape[1], D)


# speedup vs baseline: 1.0184x; 1.0184x over previous
import functools

import jax
import jax.numpy as jnp
from jax import lax
from jax.experimental import pallas as pl
from jax.experimental.pallas import tpu as pltpu
from jax.experimental.pallas import tpu_sc as plsc

SCALE = 8.0


@functools.lru_cache(maxsize=None)
def _make_lookup(R, T, V, D):
    info = plsc.get_sparse_core_info()
    NC, NS, L = info.num_cores, info.num_subcores, info.num_lanes
    NW = NC * NS
    assert R % NW == 0 and D % L == 0 and T % 8 == 0
    r_per_w = R // NW
    NBUF = 4
    assert r_per_w % NBUF == 0
    n_outer = r_per_w // NBUF
    n_vregs = D // L

    mesh = plsc.VectorSubcoreMesh(core_axis_name="c", subcore_axis_name="s")

    @functools.partial(
        pl.kernel,
        mesh=mesh,
        compiler_params=pltpu.CompilerParams(use_tc_tiling_on_sc=False),
        out_type=jax.ShapeDtypeStruct((R, T, D), jnp.float32),
        scratch_types=[
            pltpu.VMEM((r_per_w, T), jnp.int32),
            tuple(pltpu.VMEM((T, D), jnp.float32) for _ in range(NBUF)),
            tuple(pltpu.SemaphoreType.DMA for _ in range(NBUF)),
            tuple(pltpu.SemaphoreType.DMA for _ in range(NBUF)),
        ],
    )
    def lookup(x_hbm, table_hbm, out_hbm, idx_all, row_bufs, gsems, ssems):
        wid = lax.axis_index("s") * NC + lax.axis_index("c")
        base = wid * r_per_w

        pltpu.sync_copy(x_hbm.at[pl.ds(base, r_per_w)], idx_all)

        def start_gather(c, b):
            pltpu.make_async_copy(
                table_hbm.at[idx_all.at[c]], row_bufs[b], gsems[b]
            ).start()

        for b in range(NBUF):
            start_gather(b, b)

        def outer(i, carry):
            for b in range(NBUF):
                c = i * NBUF + b
                pltpu.make_async_copy(
                    table_hbm.at[idx_all.at[c]], row_bufs[b], gsems[b]
                ).wait()

                def scale_rows(t, carry2):
                    for j in range(n_vregs):
                        row_bufs[b][t, pl.ds(j * L, L)] = (
                            row_bufs[b][t, pl.ds(j * L, L)] * SCALE
                        )
                    return carry2

                lax.fori_loop(0, T, scale_rows, 0, unroll=4)

                store = pltpu.make_async_copy(
                    row_bufs[b], out_hbm.at[base + c], ssems[b]
                )
                store.start()

                @pl.when(c + NBUF < r_per_w)
                def _():
                    store.wait()
                    start_gather(c + NBUF, b)

            return carry

        lax.fori_loop(0, n_outer, outer, 0)

        for b in range(NBUF):
            pltpu.make_async_copy(
                row_bufs[b], out_hbm.at[base], ssems[b]
            ).wait()

    return lookup


def kernel(x, table):
    R, T = x.shape
    V, D = table.shape
    return _make_lookup(R, T, V, D)(x, table)
